# baseline (speedup 1.0000x reference)
import functools

import jax
import jax.numpy as jnp
from jax import lax
from jax.experimental import pallas as pl
from jax.experimental.pallas import tpu as pltpu
from jax.experimental.pallas import tpu_sc as plsc

B = 1024
F = 26
V = 1000
D = 1024

NC = 2
NS = 16
NW = NC * NS
N = B * F
BPW = N // NW
C = 32
NCH = BPW // C
LANES = 16


def _body(table_hbm, xflat_hbm, out_hbm, idx_v, buf0, buf1, gsem0, gsem1,
          wsem0, wsem1):
    wid = lax.axis_index("s") * NC + lax.axis_index("c")
    base = wid * BPW

    pltpu.sync_copy(xflat_hbm.at[pl.ds(base, BPW)], idx_v)
    for j in range(BPW // LANES):
        off = ((lax.iota(jnp.int32, LANES) + (j * LANES) % F) % F) * V
        idx_v[pl.ds(j * LANES, LANES)] = idx_v[pl.ds(j * LANES, LANES)] + off

    bufs = (buf0, buf1)
    gsems = (gsem0, gsem1)
    wsems = (wsem0, wsem1)

    def start_gather(chunk, b):
        pltpu.async_copy(table_hbm.at[idx_v.at[pl.ds(chunk * C, C)]],
                         bufs[b], gsems[b])

    def wait_gather(b):
        pltpu.make_async_copy(out_hbm.at[pl.ds(0, C)], bufs[b],
                              gsems[b]).wait()

    def start_write(chunk, b):
        pltpu.async_copy(bufs[b], out_hbm.at[pl.ds(base + chunk * C, C)],
                         wsems[b])

    def wait_write(b):
        pltpu.make_async_copy(bufs[b], out_hbm.at[pl.ds(0, C)],
                              wsems[b]).wait()

    start_gather(0, 0)
    start_gather(1, 1)
    wait_gather(0)
    start_write(0, 0)
    wait_gather(1)
    start_write(1, 1)

    @pl.loop(2, NCH, step=2)
    def _(jj):
        for b in range(2):
            wait_write(b)
            start_gather(jj + b, b)
            wait_gather(b)
            start_write(jj + b, b)

    wait_write(0)
    wait_write(1)


def _encode(table, xflat):
    mesh = plsc.VectorSubcoreMesh(core_axis_name="c", subcore_axis_name="s")
    return pl.kernel(
        _body,
        out_type=jax.ShapeDtypeStruct((N, D), jnp.float32),
        mesh=mesh,
        scratch_types=[
            pltpu.VMEM((BPW,), jnp.int32),
            pltpu.VMEM((C, D), jnp.float32),
            pltpu.VMEM((C, D), jnp.float32),
            pltpu.SemaphoreType.DMA,
            pltpu.SemaphoreType.DMA,
            pltpu.SemaphoreType.DMA,
            pltpu.SemaphoreType.DMA,
        ],
    )(table, xflat)


def kernel(x, hv_matrix):
    xflat = x.reshape(-1).astype(jnp.int32)
    table = hv_matrix.reshape(F * V, D)
    out = _encode(table, xflat)
    return out.reshape(B, F, D)


# speedup vs baseline: 1.0010x; 1.0010x over previous
import jax
import jax.numpy as jnp
from jax import lax
from jax.experimental import pallas as pl
from jax.experimental.pallas import tpu as pltpu
from jax.experimental.pallas import tpu_sc as plsc

B = 1024
F = 26
V = 1000
D = 1024

NC = 2
NS = 16
NW = NC * NS
N = B * F
BPW = N // NW
C = 16
NCH = BPW // C
NBUF = 4
LAG = 2
LANES = 16


def _body(table_hbm, xflat_hbm, out_hbm, idx_v, bufs, gsems, wsems):
    wid = lax.axis_index("s") * NC + lax.axis_index("c")
    base = wid * BPW

    pltpu.sync_copy(xflat_hbm.at[pl.ds(base, BPW)], idx_v)
    for j in range(BPW // LANES):
        off = ((lax.iota(jnp.int32, LANES) + (j * LANES) % F) % F) * V
        idx_v[pl.ds(j * LANES, LANES)] = idx_v[pl.ds(j * LANES, LANES)] + off

    def start_gather(chunk, b):
        pltpu.async_copy(table_hbm.at[idx_v.at[pl.ds(chunk * C, C)]],
                         bufs[b], gsems[b])

    def wait_gather(b):
        pltpu.make_async_copy(out_hbm.at[pl.ds(0, C)], bufs[b],
                              gsems[b]).wait()

    def start_write(chunk, b):
        pltpu.async_copy(bufs[b], out_hbm.at[pl.ds(base + chunk * C, C)],
                         wsems[b])

    def wait_write(b):
        pltpu.make_async_copy(bufs[b], out_hbm.at[pl.ds(0, C)],
                              wsems[b]).wait()

    start_gather(0, 0)
    start_gather(1, 1)
    start_gather(2, 2)
    wait_gather(0)
    start_write(0, 0)
    start_gather(3, 3)
    wait_gather(1)
    start_write(1, 1)

    @pl.loop(NBUF, NCH, step=NBUF)
    def _(c0):
        for k in range(NBUF):
            wait_write(k)
            start_gather(c0 + k, k)
            wait_gather((k + LAG) % NBUF)
            start_write(c0 + k - LAG, (k + LAG) % NBUF)

    wait_gather(NBUF - LAG)
    start_write(NCH - LAG, NBUF - LAG)
    wait_gather(NBUF - 1)
    start_write(NCH - 1, NBUF - 1)
    for k in range(NBUF):
        wait_write(k)


def _encode(table, xflat):
    mesh = plsc.VectorSubcoreMesh(core_axis_name="c", subcore_axis_name="s")
    return pl.kernel(
        _body,
        out_type=jax.ShapeDtypeStruct((N, D), jnp.float32),
        mesh=mesh,
        scratch_types=[
            pltpu.VMEM((BPW,), jnp.int32),
            tuple(pltpu.VMEM((C, D), jnp.float32) for _ in range(NBUF)),
            tuple(pltpu.SemaphoreType.DMA for _ in range(NBUF)),
            tuple(pltpu.SemaphoreType.DMA for _ in range(NBUF)),
        ],
    )(table, xflat)


def kernel(x, hv_matrix):
    xflat = x.reshape(-1).astype(jnp.int32)
    table = hv_matrix.reshape(F * V, D)
    out = _encode(table, xflat)
    return out.reshape(B, F, D)


# speedup vs baseline: 2.8501x; 2.8474x over previous
import jax
import jax.numpy as jnp
from jax import lax
from jax.experimental import pallas as pl
from jax.experimental.pallas import tpu as pltpu
from jax.experimental.pallas import tpu_sc as plsc

B = 1024
F = 26
V = 1000
D = 1024

NC = 2
NS = 16
NW = NC * NS
N = B * F
BPW = N // NW
C = 16
NCH = BPW // C
NBUF = 4
LAG = 2
LANES = 16


def _body(table_hbm, xflat_hbm, out_hbm, idx_v, bufs, gsems, wsems):
    wid = lax.axis_index("s") * NC + lax.axis_index("c")
    base = wid * BPW

    pltpu.sync_copy(xflat_hbm.at[pl.ds(base, BPW)], idx_v)
    for j in range(BPW // LANES):
        q = base + j * LANES + lax.iota(jnp.int32, LANES)
        off = lax.shift_right_logical(q, 10) * V
        idx_v[pl.ds(j * LANES, LANES)] = idx_v[pl.ds(j * LANES, LANES)] + off

    def start_gather(chunk, b):
        pltpu.async_copy(table_hbm.at[idx_v.at[pl.ds(chunk * C, C)]],
                         bufs[b], gsems[b])

    def wait_gather(b):
        pltpu.make_async_copy(out_hbm.at[pl.ds(0, C)], bufs[b],
                              gsems[b]).wait()

    def start_write(chunk, b):
        pltpu.async_copy(bufs[b], out_hbm.at[pl.ds(base + chunk * C, C)],
                         wsems[b])

    def wait_write(b):
        pltpu.make_async_copy(bufs[b], out_hbm.at[pl.ds(0, C)],
                              wsems[b]).wait()

    start_gather(0, 0)
    start_gather(1, 1)
    start_gather(2, 2)
    wait_gather(0)
    start_write(0, 0)
    start_gather(3, 3)
    wait_gather(1)
    start_write(1, 1)

    @pl.loop(NBUF, NCH, step=NBUF)
    def _(c0):
        for k in range(NBUF):
            wait_write(k)
            start_gather(c0 + k, k)
            wait_gather((k + LAG) % NBUF)
            start_write(c0 + k - LAG, (k + LAG) % NBUF)

    wait_gather(NBUF - LAG)
    start_write(NCH - LAG, NBUF - LAG)
    wait_gather(NBUF - 1)
    start_write(NCH - 1, NBUF - 1)
    for k in range(NBUF):
        wait_write(k)


def _encode(table, xflat):
    mesh = plsc.VectorSubcoreMesh(core_axis_name="c", subcore_axis_name="s")
    return pl.kernel(
        _body,
        out_type=jax.ShapeDtypeStruct((N, D), jnp.float32),
        mesh=mesh,
        scratch_types=[
            pltpu.VMEM((BPW,), jnp.int32),
            tuple(pltpu.VMEM((C, D), jnp.float32) for _ in range(NBUF)),
            tuple(pltpu.SemaphoreType.DMA for _ in range(NBUF)),
            tuple(pltpu.SemaphoreType.DMA for _ in range(NBUF)),
        ],
    )(table, xflat)


def kernel(x, hv_matrix):
    xflat = jnp.transpose(x).reshape(-1).astype(jnp.int32)
    table = hv_matrix.reshape(F * V, D)
    out = _encode(table, xflat)
    return jnp.transpose(out.reshape(F, B, D), (1, 0, 2))


# speedup vs baseline: 2.8852x; 1.0123x over previous
import jax
import jax.numpy as jnp
from jax import lax
from jax.experimental import pallas as pl
from jax.experimental.pallas import tpu as pltpu
from jax.experimental.pallas import tpu_sc as plsc

B = 1024
F = 26
V = 1000
D = 1024

NC = 2
NS = 16
NW = NC * NS
N = B * F
BW = B // NW
C = 16
NCH = F * BW // C
NBUF = 4
LAG = 2
LANES = 16


def _body(table_hbm, xt_hbm, out_hbm, idx_v, bufs, gsems, wsems):
    wid = lax.axis_index("s") * NC + lax.axis_index("c")
    b0 = wid * BW

    blk = pl.multiple_of((wid // 4) * 128, 128)
    co = (wid % 4) * BW
    pltpu.sync_copy(xt_hbm.at[:, pl.ds(blk, 128)], idx_v)
    for f in range(F):
        for h in range(BW // LANES):
            sl = pl.ds(co + h * LANES, LANES)
            idx_v[f, sl] = idx_v[f, sl] + f * V

    def start_gather(f, h, b):
        pltpu.async_copy(table_hbm.at[idx_v.at[f, pl.ds(co + h * LANES, C)]],
                         bufs[b], gsems[b])

    def wait_gather(b):
        pltpu.make_async_copy(out_hbm.at[pl.ds(0, C)], bufs[b],
                              gsems[b]).wait()

    def start_write(f, h, b):
        row = f * B + b0 + h * C
        pltpu.async_copy(bufs[b], out_hbm.at[pl.ds(row, C)], wsems[b])

    def wait_write(b):
        pltpu.make_async_copy(bufs[b], out_hbm.at[pl.ds(0, C)],
                              wsems[b]).wait()

    start_gather(0, 0, 0)
    start_gather(0, 1, 1)
    start_gather(1, 0, 2)
    wait_gather(0)
    start_write(0, 0, 0)
    start_gather(1, 1, 3)
    wait_gather(1)
    start_write(0, 1, 1)

    @pl.loop(NBUF, NCH, step=NBUF)
    def _(c0):
        for k in range(NBUF):
            c = c0 + k
            f = lax.shift_right_logical(c, 1)
            fw = lax.shift_right_logical(c - LAG, 1)
            wait_write(k)
            start_gather(f, k % 2, k)
            wait_gather((k + LAG) % NBUF)
            start_write(fw, (k - LAG) % 2, (k + LAG) % NBUF)

    wait_gather(NBUF - LAG)
    start_write(F - 1, 0, NBUF - LAG)
    wait_gather(NBUF - 1)
    start_write(F - 1, 1, NBUF - 1)
    for k in range(NBUF):
        wait_write(k)


def _encode(table, xt):
    mesh = plsc.VectorSubcoreMesh(core_axis_name="c", subcore_axis_name="s")
    return pl.kernel(
        _body,
        out_type=jax.ShapeDtypeStruct((N, D), jnp.float32),
        mesh=mesh,
        scratch_types=[
            pltpu.VMEM((F, 128), jnp.int32),
            tuple(pltpu.VMEM((C, D), jnp.float32) for _ in range(NBUF)),
            tuple(pltpu.SemaphoreType.DMA for _ in range(NBUF)),
            tuple(pltpu.SemaphoreType.DMA for _ in range(NBUF)),
        ],
    )(table, xt)


def kernel(x, hv_matrix):
    xt = jnp.transpose(x).astype(jnp.int32)
    table = hv_matrix.reshape(F * V, D)
    out = _encode(table, xt)
    return jnp.transpose(out.reshape(F, B, D), (1, 0, 2))
